# Initial kernel scaffold; baseline (speedup 1.0000x reference)
import jax
import jax.numpy as jnp
from jax.experimental import pallas as pl


def kernel(batch):
    raise NotImplementedError("write your pallas kernel here")


# speedup vs baseline: 1.6069x; 1.6069x over previous
import jax
import jax.numpy as jnp
from jax.experimental import pallas as pl

_N = 16384
_K = 8192


def _select_mean_body(x_ref, o_ref):
    x = x_ref[...]
    b = jax.lax.bitcast_convert_type(x, jnp.int32)
    sk = jnp.where(b >= 0, b, b ^ jnp.int32(0x7FFFFFFF))

    int_min = jnp.int32(-(2**31))

    def step(i, cand):
        bit = jax.lax.shift_left(jnp.int32(1), jnp.int32(31) - i)
        cand_try = cand | bit
        thr = cand_try ^ int_min
        cnt = jnp.sum((sk >= thr).astype(jnp.int32))
        return jnp.where(cnt >= _K, cand_try, cand)

    cand = jax.lax.fori_loop(0, 32, step, jnp.int32(0))
    t = cand ^ int_min

    gt = sk > t
    cnt_gt = jnp.sum(gt.astype(jnp.int32))
    sum_gt = jnp.sum(jnp.where(gt, x, jnp.float32(0.0)))
    tb = jnp.where(t >= 0, t, t ^ jnp.int32(0x7FFFFFFF))
    val_t = jax.lax.bitcast_convert_type(tb, jnp.float32)
    out = (sum_gt + (_K - cnt_gt).astype(jnp.float32) * val_t) / jnp.float32(_K)
    o_ref[...] = jnp.full((8, 128), out, jnp.float32)


def kernel(batch):
    x2d = batch.reshape(128, 128)
    out = pl.pallas_call(
        _select_mean_body,
        out_shape=jax.ShapeDtypeStruct((8, 128), jnp.float32),
    )(x2d)
    return out[0, 0]


# speedup vs baseline: 2.2986x; 1.4305x over previous
import jax
import jax.numpy as jnp
from jax import lax
from jax.experimental import pallas as pl

_N = 16384
_K = 8192
_INT_MIN = -(2**31)


def _select_mean_body(x_ref, o_ref):
    x = x_ref[...]
    b = lax.bitcast_convert_type(x, jnp.int32)
    sk = jnp.where(b >= 0, b, b ^ jnp.int32(0x7FFFFFFF))
    uk = sk ^ jnp.int32(_INT_MIN)

    prefix = jnp.int32(0)
    need = jnp.int32(_K)
    for p in range(8):
        shift = 28 - 4 * p
        nib = lax.shift_right_logical(uk, jnp.int32(shift)) & jnp.int32(15)
        if p == 0:
            active = jnp.ones(nib.shape, jnp.int32)
        else:
            hi = lax.shift_right_logical(uk, jnp.int32(shift + 4))
            active = (hi == prefix).astype(jnp.int32)
        parts = []
        for v in range(16):
            m = jnp.where(nib == v, active, jnp.int32(0))
            parts.append(jnp.sum(m, axis=0))
        t2 = jnp.stack(parts, axis=0)
        tcnt = jnp.sum(t2, axis=1)
        se = jnp.int32(0)
        bstar = jnp.int32(0)
        cnt_above = jnp.int32(0)
        found = jnp.int32(0)
        for v in range(15, -1, -1):
            tb = tcnt[v]
            cond = jnp.logical_and(found == 0, se + tb >= need)
            bstar = jnp.where(cond, jnp.int32(v), bstar)
            cnt_above = jnp.where(cond, se, cnt_above)
            found = jnp.where(cond, jnp.int32(1), found)
            se = se + tb
        need = need - cnt_above
        prefix = (prefix << jnp.int32(4)) | bstar

    t_sk = prefix ^ jnp.int32(_INT_MIN)
    gt = sk > t_sk
    sum_gt = jnp.sum(jnp.where(gt, x, jnp.float32(0.0)))
    tb_ = jnp.where(t_sk >= 0, t_sk, t_sk ^ jnp.int32(0x7FFFFFFF))
    val_t = lax.bitcast_convert_type(tb_, jnp.float32)
    out = (sum_gt + need.astype(jnp.float32) * val_t) / jnp.float32(_K)
    o_ref[...] = jnp.full((8, 128), out, jnp.float32)


def kernel(batch):
    x2d = batch.reshape(128, 128)
    out = pl.pallas_call(
        _select_mean_body,
        out_shape=jax.ShapeDtypeStruct((8, 128), jnp.float32),
    )(x2d)
    return out[0, 0]


# speedup vs baseline: 3.2952x; 1.4336x over previous
import jax
import jax.numpy as jnp
from jax import lax
from jax.experimental import pallas as pl

_N = 16384
_K = 8192
_INT_MIN = -(2**31)


def _select_mean_body(x_ref, o_ref):
    x = x_ref[...]
    b = lax.bitcast_convert_type(x, jnp.int32)
    sk = jnp.where(b >= 0, b, b ^ jnp.int32(0x7FFFFFFF))
    uk = sk ^ jnp.int32(_INT_MIN)

    iota = lax.broadcasted_iota(jnp.int32, (16, 1), 0)
    prefix = jnp.int32(0)
    need = jnp.int32(_K)
    for p in range(8):
        shift = 28 - 4 * p
        nib = lax.shift_right_logical(uk, jnp.int32(shift)) & jnp.int32(15)
        if p == 0:
            nib_act = nib
        else:
            hi = lax.shift_right_logical(uk, jnp.int32(shift + 4))
            nib_act = jnp.where(hi == prefix, nib, jnp.int32(-1))
        rows = [jnp.zeros((1, 128), jnp.int32)]
        for v in range(1, 16):
            m = (nib_act >= v).astype(jnp.int32)
            rows.append(jnp.sum(m, axis=0, keepdims=True))
        sfx = jnp.sum(jnp.concatenate(rows, axis=0), axis=1, keepdims=True)
        pos = iota > 0
        ge = jnp.logical_and(pos, sfx >= need)
        lt = jnp.logical_and(pos, sfx < need)
        bstar = jnp.sum(jnp.where(ge, jnp.int32(1), jnp.int32(0)))
        cnt_above = jnp.max(jnp.where(lt, sfx, jnp.int32(0)))
        need = need - cnt_above
        prefix = (prefix << jnp.int32(4)) | bstar

    t_sk = prefix ^ jnp.int32(_INT_MIN)
    gt = sk > t_sk
    sum_gt = jnp.sum(jnp.where(gt, x, jnp.float32(0.0)))
    tb_ = jnp.where(t_sk >= 0, t_sk, t_sk ^ jnp.int32(0x7FFFFFFF))
    val_t = lax.bitcast_convert_type(tb_, jnp.float32)
    out = (sum_gt + need.astype(jnp.float32) * val_t) / jnp.float32(_K)
    o_ref[...] = jnp.full((1, 1), out, jnp.float32)


def kernel(batch):
    x2d = batch.reshape(128, 128)
    out = pl.pallas_call(
        _select_mean_body,
        out_shape=jax.ShapeDtypeStruct((1, 1), jnp.float32),
    )(x2d)
    return out.reshape(())


# speedup vs baseline: 3.6430x; 1.1055x over previous
import jax
import jax.numpy as jnp
from jax import lax
from jax.experimental import pallas as pl

_N = 16384
_K = 8192
_INT_MIN = -(2**31)


def _hist16(nib_act):
    one = jnp.int32(1)
    amt = lax.shift_left(nib_act & jnp.int32(7), jnp.int32(2))
    w = lax.shift_left(one, amt)
    is_lo = jnp.logical_and(nib_act >= 0, nib_act < 8)
    is_hi = nib_act >= 8
    zero = jnp.int32(0)
    words = []
    for m in (is_lo, is_hi):
        wv = jnp.where(m, w, zero)
        h1 = wv[0:8] + wv[8:16]
        h2 = wv[16:24] + wv[24:32]
        h3 = wv[32:40] + wv[40:48]
        h4 = wv[48:56] + wv[56:64]
        q1 = h1 + h2
        q2 = h3 + h4
        a1 = q1 + q2
        h5 = wv[64:72] + wv[72:80]
        h6 = wv[80:88] + wv[88:96]
        h7 = wv[96:104] + wv[104:112]
        h8 = wv[112:120] + wv[120:128]
        q3 = h5 + h6
        q4 = h7 + h8
        a2 = q3 + q4
        mask4 = jnp.int32(0x0F0F0F0F)
        ev = (a1 & mask4) + (a2 & mask4)
        od = (lax.shift_right_logical(a1, jnp.int32(4)) & mask4) + (
            lax.shift_right_logical(a2, jnp.int32(4)) & mask4
        )
        ev = jnp.sum(ev, axis=0, keepdims=True)
        od = jnp.sum(od, axis=0, keepdims=True)
        mask8 = jnp.int32(0x00FF00FF)
        rows = [
            ev & mask8,
            lax.shift_right_logical(ev, jnp.int32(8)) & mask8,
            od & mask8,
            lax.shift_right_logical(od, jnp.int32(8)) & mask8,
        ]
        words.append([jnp.sum(r) for r in rows])

    mask16 = jnp.int32(0xFFFF)
    h = [None] * 16
    for g, packs in enumerate(words):
        for r, s in enumerate(packs):
            lo_bin = (0, 2, 1, 3)[r]
            h[g * 8 + lo_bin] = s & mask16
            h[g * 8 + lo_bin + 4] = lax.shift_right_logical(s, jnp.int32(16))
    return h


def _select_mean_body(x_ref, o_ref):
    x = x_ref[...]
    b = lax.bitcast_convert_type(x, jnp.int32)
    sk = jnp.where(b >= 0, b, b ^ jnp.int32(0x7FFFFFFF))
    uk = sk ^ jnp.int32(_INT_MIN)

    prefix = jnp.int32(0)
    need = jnp.int32(_K)
    for p in range(8):
        shift = 28 - 4 * p
        nib = lax.shift_right_logical(uk, jnp.int32(shift)) & jnp.int32(15)
        if p == 0:
            nib_act = nib
        else:
            hi = lax.shift_right_logical(uk, jnp.int32(shift + 4))
            nib_act = jnp.where(hi == prefix, nib, jnp.int32(-1))
        h = _hist16(nib_act)
        sfx = [None] * 16
        run = h[15]
        sfx[15] = run
        for v in range(14, 0, -1):
            run = run + h[v]
            sfx[v] = run
        zero = jnp.int32(0)
        bstar = zero
        cnt_above = zero
        for v in range(1, 16):
            bstar = bstar + jnp.where(sfx[v] >= need, jnp.int32(1), zero)
            cnt_above = jnp.maximum(
                cnt_above, jnp.where(sfx[v] < need, sfx[v], zero)
            )
        need = need - cnt_above
        prefix = (prefix << jnp.int32(4)) | bstar

    t_sk = prefix ^ jnp.int32(_INT_MIN)
    gt = sk > t_sk
    sum_gt = jnp.sum(jnp.where(gt, x, jnp.float32(0.0)))
    tb_ = jnp.where(t_sk >= 0, t_sk, t_sk ^ jnp.int32(0x7FFFFFFF))
    val_t = lax.bitcast_convert_type(tb_, jnp.float32)
    out = (sum_gt + need.astype(jnp.float32) * val_t) / jnp.float32(_K)
    o_ref[...] = jnp.full((1, 1), out, jnp.float32)


def kernel(batch):
    x2d = batch.reshape(128, 128)
    out = pl.pallas_call(
        _select_mean_body,
        out_shape=jax.ShapeDtypeStruct((1, 1), jnp.float32),
    )(x2d)
    return out.reshape(())


# speedup vs baseline: 3.7631x; 1.0330x over previous
import jax
import jax.numpy as jnp
from jax import lax
from jax.experimental import pallas as pl

_N = 16384
_K = 8192
_INT_MIN = -(2**31)


def _onehot_words(nib):
    one = jnp.int32(1)
    amt = lax.shift_left(nib & jnp.int32(7), jnp.int32(2))
    w = lax.shift_left(one, amt)
    zero = jnp.int32(0)
    wa = jnp.where(nib < 8, w, zero)
    wb = jnp.where(nib >= 8, w, zero)
    return wa, wb


def _hist16(wa, wb, act):
    zero = jnp.int32(0)
    words = []
    for wfull in (wa, wb):
        wv = wfull if act is None else jnp.where(act, wfull, zero)
        h1 = wv[0:8] + wv[8:16]
        h2 = wv[16:24] + wv[24:32]
        h3 = wv[32:40] + wv[40:48]
        h4 = wv[48:56] + wv[56:64]
        q1 = h1 + h2
        q2 = h3 + h4
        a1 = q1 + q2
        h5 = wv[64:72] + wv[72:80]
        h6 = wv[80:88] + wv[88:96]
        h7 = wv[96:104] + wv[104:112]
        h8 = wv[112:120] + wv[120:128]
        q3 = h5 + h6
        q4 = h7 + h8
        a2 = q3 + q4
        mask4 = jnp.int32(0x0F0F0F0F)
        ev = (a1 & mask4) + (a2 & mask4)
        od = (lax.shift_right_logical(a1, jnp.int32(4)) & mask4) + (
            lax.shift_right_logical(a2, jnp.int32(4)) & mask4
        )
        ev = jnp.sum(ev, axis=0, keepdims=True)
        od = jnp.sum(od, axis=0, keepdims=True)
        mask8 = jnp.int32(0x00FF00FF)
        rows = [
            ev & mask8,
            lax.shift_right_logical(ev, jnp.int32(8)) & mask8,
            od & mask8,
            lax.shift_right_logical(od, jnp.int32(8)) & mask8,
        ]
        words.append([jnp.sum(r) for r in rows])

    mask16 = jnp.int32(0xFFFF)
    h = [None] * 16
    for g, packs in enumerate(words):
        for r, s in enumerate(packs):
            lo_bin = (0, 2, 1, 3)[r]
            h[g * 8 + lo_bin] = s & mask16
            h[g * 8 + lo_bin + 4] = lax.shift_right_logical(s, jnp.int32(16))
    return h


def _select_mean_body(x_ref, o_ref):
    x = x_ref[...]
    b = lax.bitcast_convert_type(x, jnp.int32)
    sk = jnp.where(b >= 0, b, b ^ jnp.int32(0x7FFFFFFF))
    uk = sk ^ jnp.int32(_INT_MIN)

    prefix = jnp.int32(0)
    need = jnp.int32(_K)
    nib0 = lax.shift_right_logical(uk, jnp.int32(28))
    nxt = _onehot_words(nib0)
    for p in range(8):
        wa, wb = nxt
        if p == 0:
            act = None
        else:
            shift = 28 - 4 * p
            hi = lax.shift_right_logical(uk, jnp.int32(shift + 4))
            act = hi == prefix
        if p < 7:
            nshift = 28 - 4 * (p + 1)
            nnib = lax.shift_right_logical(uk, jnp.int32(nshift)) & jnp.int32(15)
            nxt = _onehot_words(nnib)
        h = _hist16(wa, wb, act)
        sfx = [None] * 16
        run = h[15]
        sfx[15] = run
        for v in range(14, 0, -1):
            run = run + h[v]
            sfx[v] = run
        zero = jnp.int32(0)
        bstar = zero
        cnt_above = zero
        for v in range(1, 16):
            bstar = bstar + jnp.where(sfx[v] >= need, jnp.int32(1), zero)
            cnt_above = jnp.maximum(
                cnt_above, jnp.where(sfx[v] < need, sfx[v], zero)
            )
        need = need - cnt_above
        prefix = (prefix << jnp.int32(4)) | bstar

    t_sk = prefix ^ jnp.int32(_INT_MIN)
    gt = sk > t_sk
    sum_gt = jnp.sum(jnp.where(gt, x, jnp.float32(0.0)))
    tb_ = jnp.where(t_sk >= 0, t_sk, t_sk ^ jnp.int32(0x7FFFFFFF))
    val_t = lax.bitcast_convert_type(tb_, jnp.float32)
    out = (sum_gt + need.astype(jnp.float32) * val_t) / jnp.float32(_K)
    o_ref[...] = jnp.full((1, 1), out, jnp.float32)


def kernel(batch):
    x2d = batch.reshape(128, 128)
    out = pl.pallas_call(
        _select_mean_body,
        out_shape=jax.ShapeDtypeStruct((1, 1), jnp.float32),
    )(x2d)
    return out.reshape(())
